# baseline (speedup 1.0000x reference)
import functools

import jax
import jax.numpy as jnp
from jax import lax
from jax.experimental import pallas as pl
from jax.experimental.pallas import tpu as pltpu
from jax.experimental.pallas import tpu_sc as plsc

V = 1000000
D = 64
B = 16384
NN = 5
NW = 32
BPW = B // NW
NCH = BPW // 128
CHK = 64
NSLAB = V // 16


def _sc_kernel():
    mesh = plsc.VectorSubcoreMesh(core_axis_name="c", subcore_axis_name="s")

    @functools.partial(
        pl.kernel,
        mesh=mesh,
        compiler_params=pltpu.CompilerParams(needs_layout_passes=False),
        out_type=jax.ShapeDtypeStruct((NN, B // 128, 128), jnp.float32),
        scratch_types=[
            pltpu.VMEM((NCH, 128), jnp.int32),
            pltpu.VMEM((NN * NCH, 128), jnp.int32),
            pltpu.VMEM((CHK, 8, 128), jnp.bfloat16),
            pltpu.VMEM((CHK, 8, 128), jnp.bfloat16),
            pltpu.VMEM((NN, NCH, 128), jnp.float32),
            pltpu.SemaphoreType.DMA,
        ],
    )
    def k(tgt_hbm, ctx_hbm, wt_hbm, wc_hbm, out_hbm, idx_t, idx_c,
          slabs_t, slabs_c, dots_v, sem):
        wid = lax.axis_index("s") * 2 + lax.axis_index("c")
        crow = wid * NCH
        lanes = lax.iota(jnp.int32, 16)

        pltpu.sync_copy(tgt_hbm.at[pl.ds(crow, NCH)], idx_t)
        for n in range(NN):
            pltpu.sync_copy(ctx_hbm.at[n, pl.ds(crow, NCH)],
                            idx_c.at[pl.ds(n * NCH, NCH)])

        def issue(table, idx_ref, slab_ref, j, roff):
            def body(g, _):
                s = j * CHK + g * 16
                v = idx_ref[roff + (s >> 7), pl.ds(s & 127, 16)] >> 4
                for i in range(16):
                    pltpu.async_copy(
                        table.at[pl.ds(v[i], 1)],
                        slab_ref.at[pl.ds(g * 16 + i, 1)], sem)
                return _
            lax.fori_loop(0, CHK // 16, body, 0)

        def drain(table, slab_ref):
            pltpu.make_async_copy(
                table.at[pl.ds(0, CHK)], slab_ref, sem).wait()

        slabs_t32 = slabs_t.bitcast(jnp.int32)
        slabs_c32 = slabs_c.bitcast(jnp.int32)

        def dot_group(n, j, g, _):
            s = j * CHK + g * 16
            vt = idx_t[s >> 7, pl.ds(s & 127, 16)]
            vc = idx_c[n * NCH + (s >> 7), pl.ds(s & 127, 16)]
            rt_vec = ((vt >> 1) & 6) + (vt & 1)
            bt_vec = (vt >> 1) & 1
            rc_vec = ((vc >> 1) & 6) + (vc & 1)
            bc_vec = (vc >> 1) & 1
            res = jnp.zeros((16,), jnp.float32)
            for i in range(16):
                p = g * 16 + i
                rt, bt = rt_vec[i], bt_vec[i]
                rc, bc = rc_vec[i], bc_vec[i]
                acc = None
                for dc in range(D // 16):
                    ww = slabs_t32[p, rt, pl.ds(dc * 16, 16)]
                    cw = slabs_c32[p, rc, pl.ds(dc * 16, 16)]
                    wa, wb = plsc.unpack(
                        plsc.bitcast(ww, jnp.bfloat16),
                        format=plsc.PackFormat.INTERLEAVED)
                    ca, cb = plsc.unpack(
                        plsc.bitcast(cw, jnp.bfloat16),
                        format=plsc.PackFormat.INTERLEAVED)
                    we = jnp.where(bt == 0, wa, wb)
                    ce = jnp.where(bc == 0, ca, cb)
                    part = we * ce
                    acc = part if acc is None else acc + part
                res = jnp.where(lanes == i, jnp.sum(acc), res)
            dots_v[n, s >> 7, pl.ds(s & 127, 16)] = res
            return _

        def chunk_body(j, _):
            issue(wt_hbm, idx_t, slabs_t, j, 0)
            drain(wt_hbm, slabs_t)
            for n in range(NN):
                issue(wc_hbm, idx_c, slabs_c, j, n * NCH)
                drain(wc_hbm, slabs_c)
                lax.fori_loop(0, CHK // 16,
                              lambda g, c: dot_group(n, j, g, c), 0)
            return _

        lax.fori_loop(0, BPW // CHK, chunk_body, 0)
        for n in range(NN):
            pltpu.sync_copy(dots_v.at[n], out_hbm.at[n, pl.ds(crow, NCH)])

    return k


_k = _sc_kernel()


def kernel(target, context, W_target, W_context):
    tgt2 = target.reshape(B // 128, 128)
    ctx3 = context.reshape(B, NN).T.reshape(NN, B // 128, 128)
    wt16 = W_target.astype(jnp.bfloat16).reshape(NSLAB, 8, 128)
    wc16 = W_context.astype(jnp.bfloat16).reshape(NSLAB, 8, 128)
    out = _k(tgt2, ctx3, wt16, wc16)
    return out.reshape(NN, B).T


# speedup vs baseline: 1.5569x; 1.5569x over previous
import functools

import jax
import jax.numpy as jnp
from jax import lax
from jax.experimental import pallas as pl
from jax.experimental.pallas import tpu as pltpu
from jax.experimental.pallas import tpu_sc as plsc

V = 1000000
D = 64
B = 16384
NN = 5
NW = 32
BPW = B // NW
NCH = BPW // 128


def _sc_kernel():
    mesh = plsc.VectorSubcoreMesh(core_axis_name="c", subcore_axis_name="s")

    @functools.partial(
        pl.kernel,
        mesh=mesh,
        compiler_params=pltpu.CompilerParams(needs_layout_passes=False),
        out_type=jax.ShapeDtypeStruct((NN, B // 128, 128), jnp.float32),
        scratch_types=[
            pltpu.VMEM((NCH, 128), jnp.int32),
            pltpu.VMEM((NCH, 128), jnp.int32),
            pltpu.VMEM((BPW, D), jnp.float32),
            pltpu.VMEM((128, D), jnp.float32),
            pltpu.VMEM((NCH, 128), jnp.float32),
            pltpu.SemaphoreType.DMA,
        ],
    )
    def k(tgt_hbm, ctx_hbm, wt_hbm, wc_hbm, out_hbm, idx_t, idx_c,
          rows_t, rows_c, dots_v, sem):
        wid = lax.axis_index("s") * 2 + lax.axis_index("c")
        crow = wid * NCH
        lanes = lax.iota(jnp.int32, 16)

        def fetch_group(table, idx_ref, rows_ref, row_of_g):
            def body(g, _):
                v = idx_ref[g >> 3, pl.ds((g & 7) * 16, 16)]
                base = row_of_g(g)
                for i in range(16):
                    pltpu.async_copy(
                        table.at[pl.ds(v[i], 1)],
                        rows_ref.at[pl.ds(base + i, 1)], sem)
                return _
            return body

        pltpu.sync_copy(tgt_hbm.at[pl.ds(crow, NCH)], idx_t)
        lax.fori_loop(0, BPW // 16,
                      fetch_group(wt_hbm, idx_t, rows_t, lambda g: g * 16), 0)
        pltpu.make_async_copy(wt_hbm.at[pl.ds(0, BPW)], rows_t, sem).wait()

        def dot_group(j, g, _):
            res = jnp.zeros((16,), jnp.float32)
            for i in range(16):
                p = g * 16 + i
                b = j * 128 + p
                acc = None
                for dc in range(D // 16):
                    we = rows_t[b, pl.ds(dc * 16, 16)]
                    ce = rows_c[p, pl.ds(dc * 16, 16)]
                    acc = we * ce if acc is None else acc + we * ce
                res = jnp.where(lanes == i, jnp.sum(acc), res)
            dots_v[j, pl.ds(g * 16, 16)] = res
            return _

        def chunk_body(j, _):
            def issue(g, c):
                return fetch_group(wc_hbm, idx_c, rows_c,
                                   lambda gg: (gg & 7) * 16)(g, c)
            lax.fori_loop(j * 8, j * 8 + 8, issue, 0)
            pltpu.make_async_copy(
                wc_hbm.at[pl.ds(0, 128)], rows_c, sem).wait()
            lax.fori_loop(0, 128 // 16,
                          lambda g, c: dot_group(j, g, c), 0)
            return _

        for n in range(NN):
            pltpu.sync_copy(ctx_hbm.at[n, pl.ds(crow, NCH)], idx_c)
            lax.fori_loop(0, NCH, chunk_body, 0)
            pltpu.sync_copy(dots_v, out_hbm.at[n, pl.ds(crow, NCH)])

    return k


_k = _sc_kernel()


def kernel(target, context, W_target, W_context):
    tgt2 = target.reshape(B // 128, 128)
    ctx3 = context.reshape(B, NN).T.reshape(NN, B // 128, 128)
    out = _k(tgt2, ctx3, W_target, W_context)
    return out.reshape(NN, B).T
